# baseline (speedup 1.0000x reference)
import functools

import jax
import jax.numpy as jnp
from jax import lax
from jax.experimental import pallas as pl
from jax.experimental.pallas import tpu as pltpu
from jax.experimental.pallas import tpu_sc as plsc

N = 10000
E = 320000
D = 128
G = 128

NC = 2
NS = 16
NW = NC * NS
CH = 40
NCHUNK = 250
EPW = NCHUNK * CH
EPAD = NW * EPW
NBUF = 8
NP = 10240
RPS = NP // NS


def _sc_agg_body(x_hbm, src_hbm, dst_hbm, out_hbm, *sc):
    c = lax.axis_index("c")
    s = lax.axis_index("s")
    wid = s * NC + c
    sbuf = sc[0:NBUF]
    dbuf = sc[NBUF:2 * NBUF]
    rows = sc[2 * NBUF:3 * NBUF]
    acc = sc[3 * NBUF]
    gsem = sc[3 * NBUF + 1:4 * NBUF + 1]
    ssem = sc[4 * NBUF + 1:5 * NBUF + 1]
    dsem = sc[5 * NBUF + 1:6 * NBUF + 1]

    zero = jnp.zeros((16,), jnp.float32)

    def zrow(r, _):
        for col in range(D // 16):
            rows[0][r, pl.ds(col * 16, 16)] = zero
        return 0

    lax.fori_loop(0, CH, zrow, 0)
    for k in range(RPS // CH):
        pltpu.sync_copy(rows[0], acc.at[pl.ds(s * RPS + k * CH, CH)])
    plsc.subcore_barrier()

    def load_s(ch, b):
        pltpu.async_copy(src_hbm.at[pl.ds(wid * EPW + ch * CH, CH)], sbuf[b], ssem[b])

    def load_d(ch, b):
        pltpu.async_copy(dst_hbm.at[pl.ds(wid * EPW + ch * CH, CH)], dbuf[b], dsem[b])

    def wait_s(ch, b):
        pltpu.make_async_copy(
            src_hbm.at[pl.ds(wid * EPW + ch * CH, CH)], sbuf[b], ssem[b]).wait()

    def wait_d(ch, b):
        pltpu.make_async_copy(
            dst_hbm.at[pl.ds(wid * EPW + ch * CH, CH)], dbuf[b], dsem[b]).wait()

    def gather(b):
        pltpu.async_copy(x_hbm.at[sbuf[b]], rows[b], gsem[b])

    for b in range(NBUF):
        load_s(b, b)
        load_d(b, b)
    for b in range(NBUF):
        wait_s(b, b)
        gather(b)

    def group(g, _):
        for b in range(NBUF):
            ch = g + b

            @pl.when(ch < NCHUNK)
            def _():
                pltpu.make_async_copy(x_hbm.at[sbuf[b]], rows[b], gsem[b]).wait()

                @pl.when(ch + NBUF < NCHUNK)
                def _():
                    load_s(ch + NBUF, b)

                wait_d(ch, b)
                pltpu.sync_copy(rows[b], acc.at[dbuf[b]], add=True)

                @pl.when(ch + NBUF < NCHUNK)
                def _():
                    load_d(ch + NBUF, b)
                    wait_s(ch + NBUF, b)
                    gather(b)
        return 0

    lax.fori_loop(0, (NCHUNK + NBUF - 1) // NBUF, lambda g, u: group(NBUF * g, u), 0)
    plsc.subcore_barrier()

    pltpu.sync_copy(acc.at[pl.ds(s * RPS, RPS)], out_hbm.at[c, pl.ds(s * RPS, RPS)])


@functools.cache
def _sc_agg_kernel():
    return pl.kernel(
        _sc_agg_body,
        out_type=jax.ShapeDtypeStruct((NC, NP, D), jnp.float32),
        mesh=plsc.VectorSubcoreMesh(core_axis_name="c", subcore_axis_name="s"),
        scratch_types=(
            [pltpu.VMEM((CH,), jnp.int32)] * (2 * NBUF)
            + [pltpu.VMEM((CH, D), jnp.float32)] * NBUF
            + [pltpu.VMEM_SHARED((NP, D), jnp.float32)]
            + [pltpu.SemaphoreType.DMA] * (3 * NBUF)
        ),
    )


def _sc_agg(h, src, dst):
    return _sc_agg_kernel()(h, src, dst)


def _dot(a, b):
    return lax.dot(a, b, preferred_element_type=jnp.float32)


BLK = 1000
NBLK = N // BLK


def _layer1_body(x_ref, p0_ref, p1_ref, wa, ba, wb, bb, o_ref):
    z = x_ref[...] + p0_ref[0] + p1_ref[0]
    a = jnp.maximum(_dot(z, wa[...]) + ba[...], 0.0)
    o_ref[...] = jnp.maximum(_dot(a, wb[...]) + bb[...], 0.0)


def _layer1(x, partial, Wa, ba, Wb, bb):
    w_spec = pl.BlockSpec((D, D), lambda i: (0, 0))
    b_spec = pl.BlockSpec((1, D), lambda i: (0, 0))
    return pl.pallas_call(
        _layer1_body,
        grid=(NBLK,),
        in_specs=[
            pl.BlockSpec((BLK, D), lambda i: (i, 0)),
            pl.BlockSpec((1, BLK, D), lambda i: (0, i, 0)),
            pl.BlockSpec((1, BLK, D), lambda i: (1, i, 0)),
            w_spec, b_spec, w_spec, b_spec,
        ],
        out_specs=pl.BlockSpec((BLK, D), lambda i: (i, 0)),
        out_shape=jax.ShapeDtypeStruct((N, D), jnp.float32),
    )(x, partial, partial, Wa, ba.reshape(1, D), Wb, bb.reshape(1, D))


def _layer2_body(h_ref, p0_ref, p1_ref, b_ref, w2a, b2a, w2b, b2b,
                 wm1, bm1, wm2, bm2, wc, bc, o_ref, acc):
    i = pl.program_id(0)
    z = h_ref[...] + p0_ref[0] + p1_ref[0]
    a = jnp.maximum(_dot(z, w2a[...]) + b2a[...], 0.0)
    h2 = jnp.maximum(_dot(a, w2b[...]) + b2b[...], 0.0)
    seg = b_ref[0, 0, :]
    gids = lax.broadcasted_iota(jnp.int32, (G, BLK), 0)
    mask = (seg[None, :] == gids).astype(jnp.float32)
    contrib = _dot(mask, h2)

    @pl.when(i == 0)
    def _():
        acc[...] = contrib

    @pl.when(i > 0)
    def _():
        acc[...] += contrib

    @pl.when(i == NBLK - 1)
    def _():
        m1 = jnp.maximum(_dot(acc[...], wm1[...]) + bm1[...], 0.0)
        m2 = jnp.maximum(_dot(m1, wm2[...]) + bm2[...], 0.0)
        o_ref[...] = _dot(m2, wc[...]) + bc[...]


def _layer2(h, partial, batch3, W2a, b2a, W2b, b2b, Wm1, bm1, Wm2, bm2, Wc, bc):
    w_spec = pl.BlockSpec((D, D), lambda i: (0, 0))
    b_spec = pl.BlockSpec((1, D), lambda i: (0, 0))
    return pl.pallas_call(
        _layer2_body,
        grid=(NBLK,),
        in_specs=[
            pl.BlockSpec((BLK, D), lambda i: (i, 0)),
            pl.BlockSpec((1, BLK, D), lambda i: (0, i, 0)),
            pl.BlockSpec((1, BLK, D), lambda i: (1, i, 0)),
            pl.BlockSpec((1, 1, BLK), lambda i: (i, 0, 0)),
            w_spec, b_spec, w_spec, b_spec,
            w_spec, b_spec, w_spec, b_spec, w_spec, b_spec,
        ],
        out_specs=pl.BlockSpec((G, D), lambda i: (0, 0)),
        out_shape=jax.ShapeDtypeStruct((G, D), jnp.float32),
        scratch_shapes=[pltpu.VMEM((G, D), jnp.float32)],
    )(h, partial, partial, batch3,
      W2a, b2a.reshape(1, D), W2b, b2b.reshape(1, D),
      Wm1, bm1.reshape(1, D), Wm2, bm2.reshape(1, D), Wc, bc.reshape(1, D))


def kernel(x, edge_index, batch, W1a, b1a, W1b, b1b, W2a, b2a, W2b, b2b,
           Wm1, bm1, Wm2, bm2, Wc, bc):
    src = edge_index[0]
    dst = edge_index[1]
    batch3 = batch.reshape(NBLK, 1, BLK)

    if EPAD > E:
        pad = EPAD - E
        src = jnp.concatenate([src, jnp.zeros((pad,), jnp.int32)])
        dst = jnp.concatenate([dst, jnp.full((pad,), NP - 1, jnp.int32)])

    p1 = _sc_agg(x, src, dst)
    h1 = _layer1(x, p1, W1a, b1a, W1b, b1b)
    p2 = _sc_agg(h1, src, dst)
    return _layer2(h1, p2, batch3, W2a, b2a, W2b, b2b,
                   Wm1, bm1, Wm2, bm2, Wc, bc)


# speedup vs baseline: 1.3368x; 1.3368x over previous
import functools

import jax
import jax.numpy as jnp
from jax import lax
from jax.experimental import pallas as pl
from jax.experimental.pallas import tpu as pltpu
from jax.experimental.pallas import tpu_sc as plsc

N = 10000
E = 320000
D = 128
G = 128

NC = 2
NS = 16
NW = NC * NS
CH = 80
NCHUNK = 125
EPW = NCHUNK * CH
EPAD = NW * EPW
NBUF = 4
NP = 10240
RPS = NP // NS


def _sc_agg_body(x_hbm, src_hbm, dst_hbm, out_hbm, *sc):
    c = lax.axis_index("c")
    s = lax.axis_index("s")
    wid = s * NC + c
    sbuf = sc[0:NBUF]
    dbuf = sc[NBUF:2 * NBUF]
    rows = sc[2 * NBUF:3 * NBUF]
    acc = sc[3 * NBUF]
    gsem = sc[3 * NBUF + 1:4 * NBUF + 1]
    ssem = sc[4 * NBUF + 1:5 * NBUF + 1]
    dsem = sc[5 * NBUF + 1:6 * NBUF + 1]

    zero = jnp.zeros((16,), jnp.float32)

    def zrow(r, _):
        for col in range(D // 16):
            rows[0][r, pl.ds(col * 16, 16)] = zero
        return 0

    lax.fori_loop(0, CH, zrow, 0)
    for k in range(RPS // CH):
        pltpu.sync_copy(rows[0], acc.at[pl.ds(s * RPS + k * CH, CH)])
    plsc.subcore_barrier()

    def load_s(ch, b):
        pltpu.async_copy(src_hbm.at[pl.ds(wid * EPW + ch * CH, CH)], sbuf[b], ssem[b])

    def load_d(ch, b):
        pltpu.async_copy(dst_hbm.at[pl.ds(wid * EPW + ch * CH, CH)], dbuf[b], dsem[b])

    def wait_s(ch, b):
        pltpu.make_async_copy(
            src_hbm.at[pl.ds(wid * EPW + ch * CH, CH)], sbuf[b], ssem[b]).wait()

    def wait_d(ch, b):
        pltpu.make_async_copy(
            dst_hbm.at[pl.ds(wid * EPW + ch * CH, CH)], dbuf[b], dsem[b]).wait()

    def gather(b):
        pltpu.async_copy(x_hbm.at[sbuf[b]], rows[b], gsem[b])

    for b in range(NBUF):
        load_s(b, b)
        load_d(b, b)
    for b in range(NBUF):
        wait_s(b, b)
        gather(b)

    def group(g, _):
        for b in range(NBUF):
            ch = g + b

            @pl.when(ch < NCHUNK)
            def _():
                pltpu.make_async_copy(x_hbm.at[sbuf[b]], rows[b], gsem[b]).wait()

                @pl.when(ch + NBUF < NCHUNK)
                def _():
                    load_s(ch + NBUF, b)

                wait_d(ch, b)
                pltpu.sync_copy(rows[b], acc.at[dbuf[b]], add=True)

                @pl.when(ch + NBUF < NCHUNK)
                def _():
                    load_d(ch + NBUF, b)
                    wait_s(ch + NBUF, b)
                    gather(b)
        return 0

    lax.fori_loop(0, (NCHUNK + NBUF - 1) // NBUF, lambda g, u: group(NBUF * g, u), 0)
    plsc.subcore_barrier()

    pltpu.sync_copy(acc.at[pl.ds(s * RPS, RPS)], out_hbm.at[c, pl.ds(s * RPS, RPS)])


@functools.cache
def _sc_agg_kernel():
    return pl.kernel(
        _sc_agg_body,
        out_type=jax.ShapeDtypeStruct((NC, NP, D), jnp.float32),
        mesh=plsc.VectorSubcoreMesh(core_axis_name="c", subcore_axis_name="s"),
        scratch_types=(
            [pltpu.VMEM((CH,), jnp.int32)] * (2 * NBUF)
            + [pltpu.VMEM((CH, D), jnp.float32)] * NBUF
            + [pltpu.VMEM_SHARED((NP, D), jnp.float32)]
            + [pltpu.SemaphoreType.DMA] * (3 * NBUF)
        ),
    )


def _sc_agg(h, src, dst):
    return _sc_agg_kernel()(h, src, dst)


def _dot(a, b):
    return lax.dot(a, b, preferred_element_type=jnp.float32)


BLK = 2000
NBLK = N // BLK


def _layer1_body(x_ref, p0_ref, p1_ref, wa, ba, wb, bb, o_ref):
    z = x_ref[...] + p0_ref[0] + p1_ref[0]
    a = jnp.maximum(_dot(z, wa[...]) + ba[...], 0.0)
    o_ref[...] = jnp.maximum(_dot(a, wb[...]) + bb[...], 0.0)


def _layer1(x, partial, Wa, ba, Wb, bb):
    w_spec = pl.BlockSpec((D, D), lambda i: (0, 0))
    b_spec = pl.BlockSpec((1, D), lambda i: (0, 0))
    return pl.pallas_call(
        _layer1_body,
        grid=(NBLK,),
        in_specs=[
            pl.BlockSpec((BLK, D), lambda i: (i, 0)),
            pl.BlockSpec((1, BLK, D), lambda i: (0, i, 0)),
            pl.BlockSpec((1, BLK, D), lambda i: (1, i, 0)),
            w_spec, b_spec, w_spec, b_spec,
        ],
        out_specs=pl.BlockSpec((BLK, D), lambda i: (i, 0)),
        out_shape=jax.ShapeDtypeStruct((N, D), jnp.float32),
    )(x, partial, partial, Wa, ba.reshape(1, D), Wb, bb.reshape(1, D))


def _layer2_body(h_ref, p0_ref, p1_ref, b_ref, w2a, b2a, w2b, b2b,
                 wm1, bm1, wm2, bm2, wc, bc, o_ref, acc):
    i = pl.program_id(0)
    z = h_ref[...] + p0_ref[0] + p1_ref[0]
    a = jnp.maximum(_dot(z, w2a[...]) + b2a[...], 0.0)
    h2 = jnp.maximum(_dot(a, w2b[...]) + b2b[...], 0.0)
    seg = b_ref[0, 0, :]
    gids = lax.broadcasted_iota(jnp.int32, (G, BLK), 0)
    mask = (seg[None, :] == gids).astype(jnp.float32)
    contrib = _dot(mask, h2)

    @pl.when(i == 0)
    def _():
        acc[...] = contrib

    @pl.when(i > 0)
    def _():
        acc[...] += contrib

    @pl.when(i == NBLK - 1)
    def _():
        m1 = jnp.maximum(_dot(acc[...], wm1[...]) + bm1[...], 0.0)
        m2 = jnp.maximum(_dot(m1, wm2[...]) + bm2[...], 0.0)
        o_ref[...] = _dot(m2, wc[...]) + bc[...]


def _layer2(h, partial, batch3, W2a, b2a, W2b, b2b, Wm1, bm1, Wm2, bm2, Wc, bc):
    w_spec = pl.BlockSpec((D, D), lambda i: (0, 0))
    b_spec = pl.BlockSpec((1, D), lambda i: (0, 0))
    return pl.pallas_call(
        _layer2_body,
        grid=(NBLK,),
        in_specs=[
            pl.BlockSpec((BLK, D), lambda i: (i, 0)),
            pl.BlockSpec((1, BLK, D), lambda i: (0, i, 0)),
            pl.BlockSpec((1, BLK, D), lambda i: (1, i, 0)),
            pl.BlockSpec((1, 1, BLK), lambda i: (i, 0, 0)),
            w_spec, b_spec, w_spec, b_spec,
            w_spec, b_spec, w_spec, b_spec, w_spec, b_spec,
        ],
        out_specs=pl.BlockSpec((G, D), lambda i: (0, 0)),
        out_shape=jax.ShapeDtypeStruct((G, D), jnp.float32),
        scratch_shapes=[pltpu.VMEM((G, D), jnp.float32)],
    )(h, partial, partial, batch3,
      W2a, b2a.reshape(1, D), W2b, b2b.reshape(1, D),
      Wm1, bm1.reshape(1, D), Wm2, bm2.reshape(1, D), Wc, bc.reshape(1, D))


def kernel(x, edge_index, batch, W1a, b1a, W1b, b1b, W2a, b2a, W2b, b2b,
           Wm1, bm1, Wm2, bm2, Wc, bc):
    src = edge_index[0]
    dst = edge_index[1]
    batch3 = batch.reshape(NBLK, 1, BLK)

    if EPAD > E:
        pad = EPAD - E
        src = jnp.concatenate([src, jnp.zeros((pad,), jnp.int32)])
        dst = jnp.concatenate([dst, jnp.full((pad,), NP - 1, jnp.int32)])

    p1 = _sc_agg(x, src, dst)
    h1 = _layer1(x, p1, W1a, b1a, W1b, b1b)
    p2 = _sc_agg(h1, src, dst)
    return _layer2(h1, p2, batch3, W2a, b2a, W2b, b2b,
                   Wm1, bm1, Wm2, bm2, Wc, bc)


# speedup vs baseline: 1.3522x; 1.0115x over previous
import functools

import jax
import jax.numpy as jnp
from jax import lax
from jax.experimental import pallas as pl
from jax.experimental.pallas import tpu as pltpu
from jax.experimental.pallas import tpu_sc as plsc

N = 10000
E = 320000
D = 128
G = 128

NC = 2
NS = 16
NW = NC * NS
CH = 80
NCHUNK = 125
EPW = NCHUNK * CH
EPAD = NW * EPW
NBUF = 4
NP = 10240
RPS = NP // NS


def _sc_agg_body(x_hbm, src_hbm, dst_hbm, out_hbm, *sc):
    c = lax.axis_index("c")
    s = lax.axis_index("s")
    wid = s * NC + c
    sbuf = sc[0:NBUF]
    dbuf = sc[NBUF:2 * NBUF]
    rows = sc[2 * NBUF:3 * NBUF]
    acc = sc[3 * NBUF]
    gsem = sc[3 * NBUF + 1:4 * NBUF + 1]
    ssem = sc[4 * NBUF + 1:5 * NBUF + 1]
    dsem = sc[5 * NBUF + 1:6 * NBUF + 1]

    zero = jnp.zeros((16,), jnp.float32)

    def zrow(r, _):
        for col in range(D // 16):
            rows[0][r, pl.ds(col * 16, 16)] = zero
        return 0

    lax.fori_loop(0, CH, zrow, 0)
    for k in range(RPS // CH):
        pltpu.sync_copy(rows[0], acc.at[pl.ds(s * RPS + k * CH, CH)])
    plsc.subcore_barrier()

    def load_s(ch, b):
        pltpu.async_copy(src_hbm.at[pl.ds(wid * EPW + ch * CH, CH)], sbuf[b], ssem[b])

    def load_d(ch, b):
        pltpu.async_copy(dst_hbm.at[pl.ds(wid * EPW + ch * CH, CH)], dbuf[b], dsem[b])

    def wait_s(ch, b):
        pltpu.make_async_copy(
            src_hbm.at[pl.ds(wid * EPW + ch * CH, CH)], sbuf[b], ssem[b]).wait()

    def wait_d(ch, b):
        pltpu.make_async_copy(
            dst_hbm.at[pl.ds(wid * EPW + ch * CH, CH)], dbuf[b], dsem[b]).wait()

    def gather(b):
        pltpu.async_copy(x_hbm.at[sbuf[b]], rows[b], gsem[b])

    for b in range(NBUF):
        load_s(b, b)
        load_d(b, b)
    for b in range(NBUF):
        wait_s(b, b)
        gather(b)

    def group(g, _):
        for b in range(NBUF):
            ch = g + b

            @pl.when(ch < NCHUNK)
            def _():
                pltpu.make_async_copy(x_hbm.at[sbuf[b]], rows[b], gsem[b]).wait()

                @pl.when(ch + NBUF < NCHUNK)
                def _():
                    load_s(ch + NBUF, b)

                wait_d(ch, b)
                pltpu.sync_copy(rows[b], acc.at[dbuf[b]], add=True)

                @pl.when(ch + NBUF < NCHUNK)
                def _():
                    load_d(ch + NBUF, b)
                    wait_s(ch + NBUF, b)
                    gather(b)
        return 0

    lax.fori_loop(0, (NCHUNK + NBUF - 1) // NBUF, lambda g, u: group(NBUF * g, u), 0)
    plsc.subcore_barrier()

    pltpu.sync_copy(acc.at[pl.ds(s * RPS, RPS)], out_hbm.at[c, pl.ds(s * RPS, RPS)])


@functools.cache
def _sc_agg_kernel():
    return pl.kernel(
        _sc_agg_body,
        out_type=jax.ShapeDtypeStruct((NC, NP, D), jnp.float32),
        mesh=plsc.VectorSubcoreMesh(core_axis_name="c", subcore_axis_name="s"),
        scratch_types=(
            [pltpu.VMEM((CH,), jnp.int32)] * (2 * NBUF)
            + [pltpu.VMEM((CH, D), jnp.float32)] * NBUF
            + [pltpu.VMEM_SHARED((NP, D), jnp.float32)]
            + [pltpu.SemaphoreType.DMA] * (3 * NBUF)
        ),
    )


def _sc_agg(h, src, dst):
    return _sc_agg_kernel()(h, src, dst)


def _dot(a, b):
    return lax.dot(a, b, preferred_element_type=jnp.float32)


BLK = 5000
NBLK = N // BLK


def _layer1_body(x_ref, p0_ref, p1_ref, wa, ba, wb, bb, o_ref):
    z = x_ref[...] + p0_ref[0] + p1_ref[0]
    a = jnp.maximum(_dot(z, wa[...]) + ba[...], 0.0)
    o_ref[...] = jnp.maximum(_dot(a, wb[...]) + bb[...], 0.0)


def _layer1(x, partial, Wa, ba, Wb, bb):
    w_spec = pl.BlockSpec((D, D), lambda i: (0, 0))
    b_spec = pl.BlockSpec((1, D), lambda i: (0, 0))
    return pl.pallas_call(
        _layer1_body,
        grid=(NBLK,),
        in_specs=[
            pl.BlockSpec((BLK, D), lambda i: (i, 0)),
            pl.BlockSpec((1, BLK, D), lambda i: (0, i, 0)),
            pl.BlockSpec((1, BLK, D), lambda i: (1, i, 0)),
            w_spec, b_spec, w_spec, b_spec,
        ],
        out_specs=pl.BlockSpec((BLK, D), lambda i: (i, 0)),
        out_shape=jax.ShapeDtypeStruct((N, D), jnp.float32),
    )(x, partial, partial, Wa, ba.reshape(1, D), Wb, bb.reshape(1, D))


def _layer2_body(h_ref, p0_ref, p1_ref, b_ref, w2a, b2a, w2b, b2b,
                 wm1, bm1, wm2, bm2, wc, bc, o_ref, acc):
    i = pl.program_id(0)
    z = h_ref[...] + p0_ref[0] + p1_ref[0]
    a = jnp.maximum(_dot(z, w2a[...]) + b2a[...], 0.0)
    h2 = jnp.maximum(_dot(a, w2b[...]) + b2b[...], 0.0)
    seg = b_ref[0, 0, :]
    gids = lax.broadcasted_iota(jnp.int32, (G, BLK), 0)
    mask = (seg[None, :] == gids).astype(jnp.float32)
    contrib = _dot(mask, h2)

    @pl.when(i == 0)
    def _():
        acc[...] = contrib

    @pl.when(i > 0)
    def _():
        acc[...] += contrib

    @pl.when(i == NBLK - 1)
    def _():
        m1 = jnp.maximum(_dot(acc[...], wm1[...]) + bm1[...], 0.0)
        m2 = jnp.maximum(_dot(m1, wm2[...]) + bm2[...], 0.0)
        o_ref[...] = _dot(m2, wc[...]) + bc[...]


def _layer2(h, partial, batch3, W2a, b2a, W2b, b2b, Wm1, bm1, Wm2, bm2, Wc, bc):
    w_spec = pl.BlockSpec((D, D), lambda i: (0, 0))
    b_spec = pl.BlockSpec((1, D), lambda i: (0, 0))
    return pl.pallas_call(
        _layer2_body,
        grid=(NBLK,),
        in_specs=[
            pl.BlockSpec((BLK, D), lambda i: (i, 0)),
            pl.BlockSpec((1, BLK, D), lambda i: (0, i, 0)),
            pl.BlockSpec((1, BLK, D), lambda i: (1, i, 0)),
            pl.BlockSpec((1, 1, BLK), lambda i: (i, 0, 0)),
            w_spec, b_spec, w_spec, b_spec,
            w_spec, b_spec, w_spec, b_spec, w_spec, b_spec,
        ],
        out_specs=pl.BlockSpec((G, D), lambda i: (0, 0)),
        out_shape=jax.ShapeDtypeStruct((G, D), jnp.float32),
        scratch_shapes=[pltpu.VMEM((G, D), jnp.float32)],
    )(h, partial, partial, batch3,
      W2a, b2a.reshape(1, D), W2b, b2b.reshape(1, D),
      Wm1, bm1.reshape(1, D), Wm2, bm2.reshape(1, D), Wc, bc.reshape(1, D))


def kernel(x, edge_index, batch, W1a, b1a, W1b, b1b, W2a, b2a, W2b, b2b,
           Wm1, bm1, Wm2, bm2, Wc, bc):
    src = edge_index[0]
    dst = edge_index[1]
    batch3 = batch.reshape(NBLK, 1, BLK)

    if EPAD > E:
        pad = EPAD - E
        src = jnp.concatenate([src, jnp.zeros((pad,), jnp.int32)])
        dst = jnp.concatenate([dst, jnp.full((pad,), NP - 1, jnp.int32)])

    p1 = _sc_agg(x, src, dst)
    h1 = _layer1(x, p1, W1a, b1a, W1b, b1b)
    p2 = _sc_agg(h1, src, dst)
    return _layer2(h1, p2, batch3, W2a, b2a, W2b, b2b,
                   Wm1, bm1, Wm2, bm2, Wc, bc)


# speedup vs baseline: 1.3528x; 1.0004x over previous
import functools

import jax
import jax.numpy as jnp
from jax import lax
from jax.experimental import pallas as pl
from jax.experimental.pallas import tpu as pltpu
from jax.experimental.pallas import tpu_sc as plsc

N = 10000
E = 320000
D = 128
G = 128

NC = 2
NS = 16
NW = NC * NS
CH = 80
NCHUNK = 125
EPW = NCHUNK * CH
EPAD = NW * EPW
NBUF = 4
NP = 10240
RPS = NP // NS


def _sc_agg_body(x_hbm, src_hbm, dst_hbm, out_hbm, *sc):
    c = lax.axis_index("c")
    s = lax.axis_index("s")
    wid = s * NC + c
    HF = CH // 2
    sbuf = sc[0:NBUF]
    dbufa = sc[NBUF:2 * NBUF]
    dbufb = sc[2 * NBUF:3 * NBUF]
    rows = sc[3 * NBUF:4 * NBUF]
    acc = sc[4 * NBUF]
    gsem = sc[4 * NBUF + 1:5 * NBUF + 1]
    ssem = sc[5 * NBUF + 1:6 * NBUF + 1]
    dsem = sc[6 * NBUF + 1:7 * NBUF + 1]
    asem = sc[7 * NBUF + 1:8 * NBUF + 1]

    def load_s(ch, b):
        pltpu.async_copy(src_hbm.at[pl.ds(wid * EPW + ch * CH, CH)], sbuf[b], ssem[b])

    def load_d(ch, b):
        base = wid * EPW + ch * CH
        pltpu.async_copy(dst_hbm.at[pl.ds(base, HF)], dbufa[b], dsem[b])
        pltpu.async_copy(dst_hbm.at[pl.ds(base + HF, HF)], dbufb[b], dsem[b])

    def wait_s(ch, b):
        pltpu.make_async_copy(
            src_hbm.at[pl.ds(wid * EPW + ch * CH, CH)], sbuf[b], ssem[b]).wait()

    def wait_d(ch, b):
        base = wid * EPW + ch * CH
        pltpu.make_async_copy(dst_hbm.at[pl.ds(base, HF)], dbufa[b], dsem[b]).wait()
        pltpu.make_async_copy(dst_hbm.at[pl.ds(base + HF, HF)], dbufb[b], dsem[b]).wait()

    def gather(b):
        pltpu.async_copy(x_hbm.at[sbuf[b]], rows[b], gsem[b])

    def scatter(b):
        pltpu.async_copy(rows[b].at[pl.ds(0, HF)], acc.at[dbufa[b]], asem[b], add=True)
        pltpu.async_copy(rows[b].at[pl.ds(HF, HF)], acc.at[dbufb[b]], asem[b], add=True)
        pltpu.make_async_copy(rows[b].at[pl.ds(0, HF)], acc.at[dbufa[b]], asem[b]).wait()
        pltpu.make_async_copy(rows[b].at[pl.ds(HF, HF)], acc.at[dbufb[b]], asem[b]).wait()

    for b in range(NBUF):
        load_s(b, b)
        load_d(b, b)

    zero = jnp.zeros((16,), jnp.float32)

    def zrow(r, _):
        for col in range(D // 16):
            rows[0][r, pl.ds(col * 16, 16)] = zero
        return 0

    lax.fori_loop(0, CH, zrow, 0)
    for k in range(RPS // CH):
        pltpu.sync_copy(rows[0], acc.at[pl.ds(s * RPS + k * CH, CH)])

    for b in range(NBUF):
        wait_s(b, b)
        gather(b)
    plsc.subcore_barrier()

    def group(g, _):
        for b in range(NBUF):
            ch = g + b

            @pl.when(ch < NCHUNK)
            def _():
                pltpu.make_async_copy(x_hbm.at[sbuf[b]], rows[b], gsem[b]).wait()

                @pl.when(ch + NBUF < NCHUNK)
                def _():
                    load_s(ch + NBUF, b)

                wait_d(ch, b)
                scatter(b)

                @pl.when(ch + NBUF < NCHUNK)
                def _():
                    load_d(ch + NBUF, b)
                    wait_s(ch + NBUF, b)
                    gather(b)
        return 0

    lax.fori_loop(0, (NCHUNK + NBUF - 1) // NBUF, lambda g, u: group(NBUF * g, u), 0)
    plsc.subcore_barrier()

    pltpu.sync_copy(acc.at[pl.ds(s * RPS, RPS)], out_hbm.at[c, pl.ds(s * RPS, RPS)])


@functools.cache
def _sc_agg_kernel():
    return pl.kernel(
        _sc_agg_body,
        out_type=jax.ShapeDtypeStruct((NC, NP, D), jnp.float32),
        mesh=plsc.VectorSubcoreMesh(core_axis_name="c", subcore_axis_name="s"),
        scratch_types=(
            [pltpu.VMEM((CH,), jnp.int32)] * NBUF
            + [pltpu.VMEM((CH // 2,), jnp.int32)] * (2 * NBUF)
            + [pltpu.VMEM((CH, D), jnp.float32)] * NBUF
            + [pltpu.VMEM_SHARED((NP, D), jnp.float32)]
            + [pltpu.SemaphoreType.DMA] * (4 * NBUF)
        ),
    )


def _sc_agg(h, src, dst):
    return _sc_agg_kernel()(h, src, dst)


def _dot(a, b):
    return lax.dot(a, b, preferred_element_type=jnp.float32)


BLK = 5000
NBLK = N // BLK


def _layer1_body(x_ref, p0_ref, p1_ref, wa, ba, wb, bb, o_ref):
    z = x_ref[...] + p0_ref[0] + p1_ref[0]
    a = jnp.maximum(_dot(z, wa[...]) + ba[...], 0.0)
    o_ref[...] = jnp.maximum(_dot(a, wb[...]) + bb[...], 0.0)


def _layer1(x, partial, Wa, ba, Wb, bb):
    w_spec = pl.BlockSpec((D, D), lambda i: (0, 0))
    b_spec = pl.BlockSpec((1, D), lambda i: (0, 0))
    return pl.pallas_call(
        _layer1_body,
        grid=(NBLK,),
        in_specs=[
            pl.BlockSpec((BLK, D), lambda i: (i, 0)),
            pl.BlockSpec((1, BLK, D), lambda i: (0, i, 0)),
            pl.BlockSpec((1, BLK, D), lambda i: (1, i, 0)),
            w_spec, b_spec, w_spec, b_spec,
        ],
        out_specs=pl.BlockSpec((BLK, D), lambda i: (i, 0)),
        out_shape=jax.ShapeDtypeStruct((N, D), jnp.float32),
    )(x, partial, partial, Wa, ba.reshape(1, D), Wb, bb.reshape(1, D))


def _layer2_body(h_ref, p0_ref, p1_ref, b_ref, w2a, b2a, w2b, b2b,
                 wm1, bm1, wm2, bm2, wc, bc, o_ref, acc):
    i = pl.program_id(0)
    z = h_ref[...] + p0_ref[0] + p1_ref[0]
    a = jnp.maximum(_dot(z, w2a[...]) + b2a[...], 0.0)
    h2 = jnp.maximum(_dot(a, w2b[...]) + b2b[...], 0.0)
    seg = b_ref[0, 0, :]
    gids = lax.broadcasted_iota(jnp.int32, (G, BLK), 0)
    mask = (seg[None, :] == gids).astype(jnp.float32)
    contrib = _dot(mask, h2)

    @pl.when(i == 0)
    def _():
        acc[...] = contrib

    @pl.when(i > 0)
    def _():
        acc[...] += contrib

    @pl.when(i == NBLK - 1)
    def _():
        m1 = jnp.maximum(_dot(acc[...], wm1[...]) + bm1[...], 0.0)
        m2 = jnp.maximum(_dot(m1, wm2[...]) + bm2[...], 0.0)
        o_ref[...] = _dot(m2, wc[...]) + bc[...]


def _layer2(h, partial, batch3, W2a, b2a, W2b, b2b, Wm1, bm1, Wm2, bm2, Wc, bc):
    w_spec = pl.BlockSpec((D, D), lambda i: (0, 0))
    b_spec = pl.BlockSpec((1, D), lambda i: (0, 0))
    return pl.pallas_call(
        _layer2_body,
        grid=(NBLK,),
        in_specs=[
            pl.BlockSpec((BLK, D), lambda i: (i, 0)),
            pl.BlockSpec((1, BLK, D), lambda i: (0, i, 0)),
            pl.BlockSpec((1, BLK, D), lambda i: (1, i, 0)),
            pl.BlockSpec((1, 1, BLK), lambda i: (i, 0, 0)),
            w_spec, b_spec, w_spec, b_spec,
            w_spec, b_spec, w_spec, b_spec, w_spec, b_spec,
        ],
        out_specs=pl.BlockSpec((G, D), lambda i: (0, 0)),
        out_shape=jax.ShapeDtypeStruct((G, D), jnp.float32),
        scratch_shapes=[pltpu.VMEM((G, D), jnp.float32)],
    )(h, partial, partial, batch3,
      W2a, b2a.reshape(1, D), W2b, b2b.reshape(1, D),
      Wm1, bm1.reshape(1, D), Wm2, bm2.reshape(1, D), Wc, bc.reshape(1, D))


def kernel(x, edge_index, batch, W1a, b1a, W1b, b1b, W2a, b2a, W2b, b2b,
           Wm1, bm1, Wm2, bm2, Wc, bc):
    src = edge_index[0]
    dst = edge_index[1]
    batch3 = batch.reshape(NBLK, 1, BLK)

    if EPAD > E:
        pad = EPAD - E
        src = jnp.concatenate([src, jnp.zeros((pad,), jnp.int32)])
        dst = jnp.concatenate([dst, jnp.full((pad,), NP - 1, jnp.int32)])

    p1 = _sc_agg(x, src, dst)
    h1 = _layer1(x, p1, W1a, b1a, W1b, b1b)
    p2 = _sc_agg(h1, src, dst)
    return _layer2(h1, p2, batch3, W2a, b2a, W2b, b2b,
                   Wm1, bm1, Wm2, bm2, Wc, bc)
